# baseline (speedup 1.0000x reference)
import functools

import jax
import jax.numpy as jnp
from jax import lax
from jax.experimental import pallas as pl
from jax.experimental.pallas import tpu as pltpu
from jax.experimental.pallas import tpu_sc as plsc

BATCH = 16384
EMB = 32
NC = 2
NS = 16
NW = NC * NS
B_PER_W = BATCH // NW


def _sc_gather(u, i, ug, ig, um, im):
    mesh = plsc.VectorSubcoreMesh(core_axis_name="c", subcore_axis_name="s")
    out_t = tuple(jax.ShapeDtypeStruct((BATCH, EMB), jnp.float32) for _ in range(4))

    @functools.partial(
        pl.kernel,
        mesh=mesh,
        out_type=out_t,
        compiler_params=pltpu.CompilerParams(use_tc_tiling_on_sc=False),
        scratch_types=[
            pltpu.VMEM((B_PER_W,), jnp.int32),
            pltpu.VMEM((B_PER_W,), jnp.int32),
            pltpu.VMEM((B_PER_W, EMB), jnp.float32),
            pltpu.VMEM((B_PER_W, EMB), jnp.float32),
            pltpu.VMEM((B_PER_W, EMB), jnp.float32),
            pltpu.VMEM((B_PER_W, EMB), jnp.float32),
            pltpu.SemaphoreType.DMA,
        ],
    )
    def k(u_hbm, i_hbm, ug_hbm, ig_hbm, um_hbm, im_hbm,
          o_gu, o_gi, o_mu, o_mi,
          uidx, iidx, bgu, bgi, bmu, bmi, sem):
        wid = lax.axis_index("s") * NC + lax.axis_index("c")
        base = wid * B_PER_W
        pltpu.sync_copy(u_hbm.at[pl.ds(base, B_PER_W)], uidx)
        pltpu.sync_copy(i_hbm.at[pl.ds(base, B_PER_W)], iidx)
        c0 = pltpu.async_copy(ug_hbm.at[uidx], bgu, sem)
        c1 = pltpu.async_copy(ig_hbm.at[iidx], bgi, sem)
        c2 = pltpu.async_copy(um_hbm.at[uidx], bmu, sem)
        c3 = pltpu.async_copy(im_hbm.at[iidx], bmi, sem)
        c0.wait()
        c1.wait()
        c2.wait()
        c3.wait()
        sl = pl.ds(base, B_PER_W)
        pltpu.sync_copy(bgu, o_gu.at[sl])
        pltpu.sync_copy(bgi, o_gi.at[sl])
        pltpu.sync_copy(bmu, o_mu.at[sl])
        pltpu.sync_copy(bmi, o_mi.at[sl])

    return k(u, i, ug, ig, um, im)


def _tc_dense(gu, gi, mu, mi, W1a, W1b, b1, W2, b2, Wfg, Wfh, bf):
    BLK = 2048
    grid = (BATCH // BLK,)

    def body(gu_r, gi_r, mu_r, mi_r, W1a_r, W1b_r, b1_r, W2_r, b2_r,
             Wfg_r, Wfh_r, bf_r, out_r):
        h = jnp.dot(mu_r[...], W1a_r[...], preferred_element_type=jnp.float32)
        h = h + jnp.dot(mi_r[...], W1b_r[...], preferred_element_type=jnp.float32)
        h = jnp.maximum(h + b1_r[...], 0.0)
        h2 = jnp.dot(h, W2_r[...], preferred_element_type=jnp.float32)
        h2 = jnp.maximum(h2 + b2_r[...], 0.0)
        gmf = gu_r[...] * gi_r[...]
        acc = jnp.sum(gmf * Wfg_r[...], axis=1) + jnp.sum(h2 * Wfh_r[...], axis=1)
        out_r[...] = acc + bf_r[0, 0]

    full = lambda s: pl.BlockSpec(s, lambda b: (0, 0))
    return pl.pallas_call(
        body,
        grid=grid,
        in_specs=[
            pl.BlockSpec((BLK, EMB), lambda b: (b, 0)),
            pl.BlockSpec((BLK, EMB), lambda b: (b, 0)),
            pl.BlockSpec((BLK, EMB), lambda b: (b, 0)),
            pl.BlockSpec((BLK, EMB), lambda b: (b, 0)),
            full((EMB, 64)),
            full((EMB, 64)),
            full((1, 64)),
            full((64, EMB)),
            full((1, EMB)),
            full((1, EMB)),
            full((1, EMB)),
            full((1, 1)),
        ],
        out_specs=pl.BlockSpec((BLK,), lambda b: (b,)),
        out_shape=jax.ShapeDtypeStruct((BATCH,), jnp.float32),
        compiler_params=pltpu.CompilerParams(
            dimension_semantics=("parallel",),
        ),
    )(gu, gi, mu, mi, W1a, W1b, b1, W2, b2, Wfg, Wfh, bf)


@jax.jit
def kernel(u, i, user_emb_gmf, item_emb_gmf, user_emb_mlp, item_emb_mlp,
           W1, b1, W2, b2, Wf, bf):
    u32 = jnp.asarray(u, jnp.int32)
    i32 = jnp.asarray(i, jnp.int32)
    gu, gi, mu, mi = _sc_gather(u32, i32, user_emb_gmf, item_emb_gmf,
                                user_emb_mlp, item_emb_mlp)
    W1a = W1[:EMB, :]
    W1b = W1[EMB:, :]
    Wfg = Wf[:EMB, 0].reshape(1, EMB)
    Wfh = Wf[EMB:, 0].reshape(1, EMB)
    out = _tc_dense(gu, gi, mu, mi, W1a, W1b, b1.reshape(1, 64), W2,
                    b2.reshape(1, EMB), Wfg, Wfh, bf.reshape(1, 1))
    return out


# speedup vs baseline: 1.4152x; 1.4152x over previous
import functools

import jax
import jax.numpy as jnp
from jax import lax
from jax.experimental import pallas as pl
from jax.experimental.pallas import tpu as pltpu
from jax.experimental.pallas import tpu_sc as plsc

BATCH = 16384
EMB = 32
NC = 2
NS = 16
NW = NC * NS
B_PER_W = BATCH // NW
CHUNK = 128


def _sc_gather(u, i, ug, ig, um, im):
    mesh = plsc.VectorSubcoreMesh(core_axis_name="c", subcore_axis_name="s")
    out_t = tuple(jax.ShapeDtypeStruct((BATCH, EMB), jnp.float32) for _ in range(4))

    @functools.partial(
        pl.kernel,
        mesh=mesh,
        out_type=out_t,
        scratch_types=[
            pltpu.VMEM((B_PER_W,), jnp.int32),
            pltpu.VMEM((B_PER_W,), jnp.int32),
            pltpu.VMEM((CHUNK, EMB), jnp.float32),
            pltpu.VMEM((CHUNK, EMB), jnp.float32),
            pltpu.VMEM((CHUNK, EMB), jnp.float32),
            pltpu.VMEM((CHUNK, EMB), jnp.float32),
            pltpu.SemaphoreType.DMA,
        ],
    )
    def k(u_hbm, i_hbm, ug_hbm, ig_hbm, um_hbm, im_hbm,
          o_gu, o_gi, o_mu, o_mi,
          uidx, iidx, bgu, bgi, bmu, bmi, sem):
        wid = lax.axis_index("s") * NC + lax.axis_index("c")
        base = wid * B_PER_W
        pltpu.sync_copy(u_hbm.at[pl.ds(base, B_PER_W)], uidx)
        pltpu.sync_copy(i_hbm.at[pl.ds(base, B_PER_W)], iidx)

        def chunk_body(c, carry):
            coff = c * CHUNK

            def body(g, carry2):
                off = g * 16
                uvec = uidx[pl.ds(coff + off, 16)]
                ivec = iidx[pl.ds(coff + off, 16)]
                for j in range(16):
                    ru = uvec[j]
                    ri = ivec[j]
                    pltpu.async_copy(ug_hbm.at[ru], bgu.at[off + j], sem)
                    pltpu.async_copy(ig_hbm.at[ri], bgi.at[off + j], sem)
                    pltpu.async_copy(um_hbm.at[ru], bmu.at[off + j], sem)
                    pltpu.async_copy(im_hbm.at[ri], bmi.at[off + j], sem)
                return carry2

            lax.fori_loop(0, CHUNK // 16, body, 0)
            pltpu.make_async_copy(ug_hbm.at[pl.ds(0, CHUNK)], bgu, sem).wait()
            pltpu.make_async_copy(ig_hbm.at[pl.ds(0, CHUNK)], bgi, sem).wait()
            pltpu.make_async_copy(um_hbm.at[pl.ds(0, CHUNK)], bmu, sem).wait()
            pltpu.make_async_copy(im_hbm.at[pl.ds(0, CHUNK)], bmi, sem).wait()
            sl = pl.ds(base + coff, CHUNK)
            pltpu.sync_copy(bgu, o_gu.at[sl])
            pltpu.sync_copy(bgi, o_gi.at[sl])
            pltpu.sync_copy(bmu, o_mu.at[sl])
            pltpu.sync_copy(bmi, o_mi.at[sl])
            return carry

        lax.fori_loop(0, B_PER_W // CHUNK, chunk_body, 0)

    return k(u, i, ug, ig, um, im)


def _tc_dense(gu, gi, mu, mi, W1a, W1b, b1, W2, b2, Wfg, Wfh, bf):
    BLK = 2048
    grid = (BATCH // BLK,)

    def body(gu_r, gi_r, mu_r, mi_r, W1a_r, W1b_r, b1_r, W2_r, b2_r,
             Wfg_r, Wfh_r, bf_r, out_r):
        h = jnp.dot(mu_r[...], W1a_r[...], preferred_element_type=jnp.float32)
        h = h + jnp.dot(mi_r[...], W1b_r[...], preferred_element_type=jnp.float32)
        h = jnp.maximum(h + b1_r[...], 0.0)
        h2 = jnp.dot(h, W2_r[...], preferred_element_type=jnp.float32)
        h2 = jnp.maximum(h2 + b2_r[...], 0.0)
        gmf = gu_r[...] * gi_r[...]
        acc = jnp.sum(gmf * Wfg_r[...], axis=1) + jnp.sum(h2 * Wfh_r[...], axis=1)
        out_r[...] = acc + bf_r[0, 0]

    full = lambda s: pl.BlockSpec(s, lambda b: (0, 0))
    return pl.pallas_call(
        body,
        grid=grid,
        in_specs=[
            pl.BlockSpec((BLK, EMB), lambda b: (b, 0)),
            pl.BlockSpec((BLK, EMB), lambda b: (b, 0)),
            pl.BlockSpec((BLK, EMB), lambda b: (b, 0)),
            pl.BlockSpec((BLK, EMB), lambda b: (b, 0)),
            full((EMB, 64)),
            full((EMB, 64)),
            full((1, 64)),
            full((64, EMB)),
            full((1, EMB)),
            full((1, EMB)),
            full((1, EMB)),
            full((1, 1)),
        ],
        out_specs=pl.BlockSpec((BLK,), lambda b: (b,)),
        out_shape=jax.ShapeDtypeStruct((BATCH,), jnp.float32),
        compiler_params=pltpu.CompilerParams(
            dimension_semantics=("parallel",),
        ),
    )(gu, gi, mu, mi, W1a, W1b, b1, W2, b2, Wfg, Wfh, bf)


@jax.jit
def kernel(u, i, user_emb_gmf, item_emb_gmf, user_emb_mlp, item_emb_mlp,
           W1, b1, W2, b2, Wf, bf):
    u32 = jnp.asarray(u, jnp.int32)
    i32 = jnp.asarray(i, jnp.int32)
    gu, gi, mu, mi = _sc_gather(u32, i32, user_emb_gmf, item_emb_gmf,
                                user_emb_mlp, item_emb_mlp)
    W1a = W1[:EMB, :]
    W1b = W1[EMB:, :]
    Wfg = Wf[:EMB, 0].reshape(1, EMB)
    Wfh = Wf[EMB:, 0].reshape(1, EMB)
    out = _tc_dense(gu, gi, mu, mi, W1a, W1b, b1.reshape(1, 64), W2,
                    b2.reshape(1, EMB), Wfg, Wfh, bf.reshape(1, 1))
    return out


# speedup vs baseline: 1.6351x; 1.1554x over previous
import functools

import jax
import jax.numpy as jnp
from jax import lax
from jax.experimental import pallas as pl
from jax.experimental.pallas import tpu as pltpu
from jax.experimental.pallas import tpu_sc as plsc

BATCH = 16384
EMB = 32
NC = 2
NS = 16
NW = NC * NS
B_PER_W = BATCH // NW
CHUNK = 128

N_ROWS = 1000000
G = 8
TBLK = 512 * G
NBLKS = 245
P_ROWS = NBLKS * 128 * G


def _tc_pack(ugT, igT, umT, imT):
    def body(a_r, b_r, c_r, d_r, oa_r, ob_r, oc_r, od_r):
        for x_r, o_r in ((a_r, oa_r), (b_r, ob_r), (c_r, oc_r), (d_r, od_r)):
            xT = x_r[...].T
            for j in range(G):
                o_r[128 * j:128 * (j + 1), :] = jnp.concatenate(
                    [xT[512 * j + 128 * q:512 * j + 128 * (q + 1)]
                     for q in range(4)], axis=1)

    in_spec = pl.BlockSpec((EMB, TBLK), lambda b: (0, b))
    out_spec = pl.BlockSpec((128 * G, 128), lambda b: (b, 0))
    outs = pl.pallas_call(
        body,
        grid=(NBLKS,),
        in_specs=[in_spec] * 4,
        out_specs=[out_spec] * 4,
        out_shape=[jax.ShapeDtypeStruct((P_ROWS, 128), jnp.float32)] * 4,
        compiler_params=pltpu.CompilerParams(
            dimension_semantics=("arbitrary",),
        ),
    )(ugT, igT, umT, imT)
    return outs


def _sc_gather(gu_idx, gi_idx, Pug, Pig, Pum, Pim):
    mesh = plsc.VectorSubcoreMesh(core_axis_name="c", subcore_axis_name="s")
    out_t = tuple(jax.ShapeDtypeStruct((BATCH, 128), jnp.float32) for _ in range(4))

    @functools.partial(
        pl.kernel,
        mesh=mesh,
        out_type=out_t,
        scratch_types=[
            pltpu.VMEM((B_PER_W,), jnp.int32),
            pltpu.VMEM((B_PER_W,), jnp.int32),
            pltpu.VMEM((CHUNK, 128), jnp.float32),
            pltpu.VMEM((CHUNK, 128), jnp.float32),
            pltpu.VMEM((CHUNK, 128), jnp.float32),
            pltpu.VMEM((CHUNK, 128), jnp.float32),
            pltpu.SemaphoreType.DMA,
        ],
    )
    def k(u_hbm, i_hbm, ug_hbm, ig_hbm, um_hbm, im_hbm,
          o_gu, o_gi, o_mu, o_mi,
          uidx, iidx, bgu, bgi, bmu, bmi, sem):
        wid = lax.axis_index("s") * NC + lax.axis_index("c")
        base = wid * B_PER_W
        pltpu.sync_copy(u_hbm.at[pl.ds(base, B_PER_W)], uidx)
        pltpu.sync_copy(i_hbm.at[pl.ds(base, B_PER_W)], iidx)

        def chunk_body(c, carry):
            coff = c * CHUNK
            usl = uidx.at[pl.ds(coff, CHUNK)]
            isl = iidx.at[pl.ds(coff, CHUNK)]
            c0 = pltpu.async_copy(ug_hbm.at[usl], bgu, sem)
            c1 = pltpu.async_copy(ig_hbm.at[isl], bgi, sem)
            c2 = pltpu.async_copy(um_hbm.at[usl], bmu, sem)
            c3 = pltpu.async_copy(im_hbm.at[isl], bmi, sem)
            c0.wait()
            c1.wait()
            c2.wait()
            c3.wait()
            sl = pl.ds(base + coff, CHUNK)
            pltpu.sync_copy(bgu, o_gu.at[sl])
            pltpu.sync_copy(bgi, o_gi.at[sl])
            pltpu.sync_copy(bmu, o_mu.at[sl])
            pltpu.sync_copy(bmi, o_mi.at[sl])
            return carry

        lax.fori_loop(0, B_PER_W // CHUNK, chunk_body, 0)

    return k(gu_idx, gi_idx, Pug, Pig, Pum, Pim)


def _tc_dense(Xgu, Xgi, Xmu, Xmi, ohu, ohi, W1a, W1b, b1, W2, b2, Wfg, Wfh, bf):
    BLK = 2048
    grid = (BATCH // BLK,)

    def sel(x, oh128):
        m = x * oh128
        return (m[:, 0:32] + m[:, 32:64]) + (m[:, 64:96] + m[:, 96:128])

    def body(gu_r, gi_r, mu_r, mi_r, ohu_r, ohi_r, W1a_r, W1b_r, b1_r,
             W2_r, b2_r, Wfg_r, Wfh_r, bf_r, out_r):
        ohu_v = ohu_r[...]
        ohi_v = ohi_r[...]
        gu = sel(gu_r[...], ohu_v)
        gi = sel(gi_r[...], ohi_v)
        mu = sel(mu_r[...], ohu_v)
        mi = sel(mi_r[...], ohi_v)
        h = jnp.dot(mu, W1a_r[...], preferred_element_type=jnp.float32)
        h = h + jnp.dot(mi, W1b_r[...], preferred_element_type=jnp.float32)
        h = jnp.maximum(h + b1_r[...], 0.0)
        h2 = jnp.dot(h, W2_r[...], preferred_element_type=jnp.float32)
        h2 = jnp.maximum(h2 + b2_r[...], 0.0)
        gmf = gu * gi
        acc = jnp.sum(gmf * Wfg_r[...], axis=1) + jnp.sum(h2 * Wfh_r[...], axis=1)
        out_r[...] = acc + bf_r[0, 0]

    full = lambda s: pl.BlockSpec(s, lambda b: (0, 0))
    return pl.pallas_call(
        body,
        grid=grid,
        in_specs=[
            pl.BlockSpec((BLK, 128), lambda b: (b, 0)),
            pl.BlockSpec((BLK, 128), lambda b: (b, 0)),
            pl.BlockSpec((BLK, 128), lambda b: (b, 0)),
            pl.BlockSpec((BLK, 128), lambda b: (b, 0)),
            pl.BlockSpec((BLK, 128), lambda b: (b, 0)),
            pl.BlockSpec((BLK, 128), lambda b: (b, 0)),
            full((EMB, 64)),
            full((EMB, 64)),
            full((1, 64)),
            full((64, EMB)),
            full((1, EMB)),
            full((1, EMB)),
            full((1, EMB)),
            full((1, 1)),
        ],
        out_specs=pl.BlockSpec((BLK,), lambda b: (b,)),
        out_shape=jax.ShapeDtypeStruct((BATCH,), jnp.float32),
        compiler_params=pltpu.CompilerParams(
            dimension_semantics=("parallel",),
        ),
    )(Xgu, Xgi, Xmu, Xmi, ohu, ohi, W1a, W1b, b1, W2, b2, Wfg, Wfh, bf)


@jax.jit
def kernel(u, i, user_emb_gmf, item_emb_gmf, user_emb_mlp, item_emb_mlp,
           W1, b1, W2, b2, Wf, bf):
    u32 = jnp.asarray(u, jnp.int32)
    i32 = jnp.asarray(i, jnp.int32)

    Pug, Pig, Pum, Pim = _tc_pack(user_emb_gmf.T, item_emb_gmf.T,
                                  user_emb_mlp.T, item_emb_mlp.T)

    def packed_idx(r):
        return ((r >> 9) << 7) | (r & 127)

    gu_idx = packed_idx(u32)
    gi_idx = packed_idx(i32)
    su = (u32 >> 7) & 3
    si = (i32 >> 7) & 3
    lane_grp = jnp.arange(128, dtype=jnp.int32)[None, :] >> 5
    ohu = (su[:, None] == lane_grp).astype(jnp.float32)
    ohi = (si[:, None] == lane_grp).astype(jnp.float32)

    Xgu, Xgi, Xmu, Xmi = _sc_gather(gu_idx, gi_idx, Pug, Pig, Pum, Pim)

    W1a = W1[:EMB, :]
    W1b = W1[EMB:, :]
    Wfg = Wf[:EMB, 0].reshape(1, EMB)
    Wfh = Wf[EMB:, 0].reshape(1, EMB)
    out = _tc_dense(Xgu, Xgi, Xmu, Xmi, ohu, ohi, W1a, W1b,
                    b1.reshape(1, 64), W2, b2.reshape(1, EMB), Wfg, Wfh,
                    bf.reshape(1, 1))
    return out


# speedup vs baseline: 3.8953x; 2.3822x over previous
import functools

import jax
import jax.numpy as jnp
from jax import lax
from jax.experimental import pallas as pl
from jax.experimental.pallas import tpu as pltpu
from jax.experimental.pallas import tpu_sc as plsc

BATCH = 16384
EMB = 32
NC = 2
NS = 16
NW = NC * NS
B_PER_W = BATCH // NW
CHUNK = 256

N_ROWS = 1000000
TBLK = 4096
NBLKS = 245
M_ROWS = NBLKS * TBLK


def _tc_mix(ugT, igT, umT, imT):
    def body(a_r, b_r, c_r, d_r, o_r):
        x4 = jnp.concatenate([a_r[...], b_r[...], c_r[...], d_r[...]], axis=0)
        o_r[...] = x4.T

    in_spec = pl.BlockSpec((EMB, TBLK), lambda b: (0, b))
    return pl.pallas_call(
        body,
        grid=(NBLKS,),
        in_specs=[in_spec] * 4,
        out_specs=pl.BlockSpec((TBLK, 128), lambda b: (b, 0)),
        out_shape=jax.ShapeDtypeStruct((M_ROWS, 128), jnp.float32),
        compiler_params=pltpu.CompilerParams(
            dimension_semantics=("arbitrary",),
        ),
    )(ugT, igT, umT, imT)


def _sc_gather(u, i, M):
    mesh = plsc.VectorSubcoreMesh(core_axis_name="c", subcore_axis_name="s")
    out_t = tuple(jax.ShapeDtypeStruct((BATCH, 128), jnp.float32) for _ in range(2))

    @functools.partial(
        pl.kernel,
        mesh=mesh,
        out_type=out_t,
        scratch_types=[
            pltpu.VMEM((B_PER_W,), jnp.int32),
            pltpu.VMEM((B_PER_W,), jnp.int32),
            pltpu.VMEM((CHUNK, 128), jnp.float32),
            pltpu.VMEM((CHUNK, 128), jnp.float32),
            pltpu.SemaphoreType.DMA,
        ],
    )
    def k(u_hbm, i_hbm, m_hbm, o_u, o_i, uidx, iidx, bu, bi, sem):
        wid = lax.axis_index("s") * NC + lax.axis_index("c")
        base = wid * B_PER_W
        pltpu.sync_copy(u_hbm.at[pl.ds(base, B_PER_W)], uidx)
        pltpu.sync_copy(i_hbm.at[pl.ds(base, B_PER_W)], iidx)

        def chunk_body(c, carry):
            coff = c * CHUNK
            c0 = pltpu.async_copy(m_hbm.at[uidx.at[pl.ds(coff, CHUNK)]], bu, sem)
            c1 = pltpu.async_copy(m_hbm.at[iidx.at[pl.ds(coff, CHUNK)]], bi, sem)
            c0.wait()
            c1.wait()
            sl = pl.ds(base + coff, CHUNK)
            pltpu.sync_copy(bu, o_u.at[sl])
            pltpu.sync_copy(bi, o_i.at[sl])
            return carry

        lax.fori_loop(0, B_PER_W // CHUNK, chunk_body, 0)

    return k(u, i, M)


def _tc_dense(Xu, Xi, W1a, W1b, b1, W2, b2, Wfg, Wfh, bf):
    BLK = 2048
    grid = (BATCH // BLK,)

    def body(xu_r, xi_r, W1a_r, W1b_r, b1_r, W2_r, b2_r,
             Wfg_r, Wfh_r, bf_r, out_r):
        xu = xu_r[...]
        xi = xi_r[...]
        gu = xu[:, 0:32]
        mu = xu[:, 64:96]
        gi = xi[:, 32:64]
        mi = xi[:, 96:128]
        h = jnp.dot(mu, W1a_r[...], preferred_element_type=jnp.float32)
        h = h + jnp.dot(mi, W1b_r[...], preferred_element_type=jnp.float32)
        h = jnp.maximum(h + b1_r[...], 0.0)
        h2 = jnp.dot(h, W2_r[...], preferred_element_type=jnp.float32)
        h2 = jnp.maximum(h2 + b2_r[...], 0.0)
        gmf = gu * gi
        acc = jnp.sum(gmf * Wfg_r[...], axis=1) + jnp.sum(h2 * Wfh_r[...], axis=1)
        out_r[...] = acc + bf_r[0, 0]

    full = lambda s: pl.BlockSpec(s, lambda b: (0, 0))
    return pl.pallas_call(
        body,
        grid=grid,
        in_specs=[
            pl.BlockSpec((BLK, 128), lambda b: (b, 0)),
            pl.BlockSpec((BLK, 128), lambda b: (b, 0)),
            full((EMB, 64)),
            full((EMB, 64)),
            full((1, 64)),
            full((64, EMB)),
            full((1, EMB)),
            full((1, EMB)),
            full((1, EMB)),
            full((1, 1)),
        ],
        out_specs=pl.BlockSpec((BLK,), lambda b: (b,)),
        out_shape=jax.ShapeDtypeStruct((BATCH,), jnp.float32),
        compiler_params=pltpu.CompilerParams(
            dimension_semantics=("parallel",),
        ),
    )(Xu, Xi, W1a, W1b, b1, W2, b2, Wfg, Wfh, bf)


@jax.jit
def kernel(u, i, user_emb_gmf, item_emb_gmf, user_emb_mlp, item_emb_mlp,
           W1, b1, W2, b2, Wf, bf):
    u32 = jnp.asarray(u, jnp.int32)
    i32 = jnp.asarray(i, jnp.int32)

    M = _tc_mix(user_emb_gmf.T, item_emb_gmf.T,
                user_emb_mlp.T, item_emb_mlp.T)
    Xu, Xi = _sc_gather(u32, i32, M)

    W1a = W1[:EMB, :]
    W1b = W1[EMB:, :]
    Wfg = Wf[:EMB, 0].reshape(1, EMB)
    Wfh = Wf[EMB:, 0].reshape(1, EMB)
    out = _tc_dense(Xu, Xi, W1a, W1b, b1.reshape(1, 64), W2,
                    b2.reshape(1, EMB), Wfg, Wfh, bf.reshape(1, 1))
    return out


# speedup vs baseline: 4.1340x; 1.0613x over previous
import functools

import jax
import jax.numpy as jnp
from jax import lax
from jax.experimental import pallas as pl
from jax.experimental.pallas import tpu as pltpu
from jax.experimental.pallas import tpu_sc as plsc

BATCH = 16384
EMB = 32
NC = 2
NS = 16
NW = NC * NS
B_PER_W = BATCH // NW
CHUNK = 256

N_ROWS = 1000000
TBLK = 4096
NBLKS = 245
M_ROWS = NBLKS * TBLK // 2


def _tc_mix(ugT, igT, umT, imT):
    def body(a_r, b_r, c_r, d_r, o_r):
        x4 = jnp.concatenate([a_r[...], b_r[...], c_r[...], d_r[...]], axis=0)
        b = jax.lax.bitcast_convert_type(x4, jnp.int32)
        bb = (b + 32767 + ((b >> 16) & 1)) >> 16
        bbT = bb.T
        lo = bbT[0:TBLK // 2]
        hi = bbT[TBLK // 2:TBLK]
        o_r[...] = (lo & 65535) | (hi << 16)

    in_spec = pl.BlockSpec((EMB, TBLK), lambda b: (0, b))
    return pl.pallas_call(
        body,
        grid=(NBLKS,),
        in_specs=[in_spec] * 4,
        out_specs=pl.BlockSpec((TBLK // 2, 128), lambda b: (b, 0)),
        out_shape=jax.ShapeDtypeStruct((M_ROWS, 128), jnp.int32),
        compiler_params=pltpu.CompilerParams(
            dimension_semantics=("arbitrary",),
        ),
    )(ugT, igT, umT, imT)


def _sc_gather(u, i, M):
    mesh = plsc.VectorSubcoreMesh(core_axis_name="c", subcore_axis_name="s")
    out_t = tuple(jax.ShapeDtypeStruct((BATCH, 128), jnp.int32) for _ in range(2))

    @functools.partial(
        pl.kernel,
        mesh=mesh,
        out_type=out_t,
        scratch_types=[
            pltpu.VMEM((B_PER_W,), jnp.int32),
            pltpu.VMEM((B_PER_W,), jnp.int32),
            pltpu.VMEM((CHUNK, 128), jnp.int32),
            pltpu.VMEM((CHUNK, 128), jnp.int32),
            pltpu.SemaphoreType.DMA,
        ],
    )
    def k(u_hbm, i_hbm, m_hbm, o_u, o_i, uidx, iidx, bu, bi, sem):
        wid = lax.axis_index("s") * NC + lax.axis_index("c")
        base = wid * B_PER_W
        pltpu.sync_copy(u_hbm.at[pl.ds(base, B_PER_W)], uidx)
        pltpu.sync_copy(i_hbm.at[pl.ds(base, B_PER_W)], iidx)

        def chunk_body(c, carry):
            coff = c * CHUNK
            c0 = pltpu.async_copy(m_hbm.at[uidx.at[pl.ds(coff, CHUNK)]], bu, sem)
            c1 = pltpu.async_copy(m_hbm.at[iidx.at[pl.ds(coff, CHUNK)]], bi, sem)
            c0.wait()
            c1.wait()
            sl = pl.ds(base + coff, CHUNK)
            pltpu.sync_copy(bu, o_u.at[sl])
            pltpu.sync_copy(bi, o_i.at[sl])
            return carry

        lax.fori_loop(0, B_PER_W // CHUNK, chunk_body, 0)

    return k(u, i, M)


def _tc_dense(Xu, Xi, mu_odd, mi_odd, W1a, W1b, b1, W2, b2, Wfg, Wfh, bf):
    BLK = 2048
    grid = (BATCH // BLK,)

    def unpack(x_i32, odd):
        bits = jnp.where(odd != 0,
                         x_i32 & jnp.int32(-65536),
                         x_i32 << 16)
        return jax.lax.bitcast_convert_type(bits, jnp.float32)

    def body(xu_r, xi_r, mou_r, moi_r, W1a_r, W1b_r, b1_r, W2_r, b2_r,
             Wfg_r, Wfh_r, bf_r, out_r):
        xu = unpack(xu_r[...], mou_r[...])
        xi = unpack(xi_r[...], moi_r[...])
        gu = xu[:, 0:32]
        mu = xu[:, 64:96]
        gi = xi[:, 32:64]
        mi = xi[:, 96:128]
        h = jnp.dot(mu, W1a_r[...], preferred_element_type=jnp.float32)
        h = h + jnp.dot(mi, W1b_r[...], preferred_element_type=jnp.float32)
        h = jnp.maximum(h + b1_r[...], 0.0)
        h2 = jnp.dot(h, W2_r[...], preferred_element_type=jnp.float32)
        h2 = jnp.maximum(h2 + b2_r[...], 0.0)
        gmf = gu * gi
        acc = jnp.sum(gmf * Wfg_r[...], axis=1) + jnp.sum(h2 * Wfh_r[...], axis=1)
        out_r[...] = acc + bf_r[0, 0]

    full = lambda s: pl.BlockSpec(s, lambda b: (0, 0))
    return pl.pallas_call(
        body,
        grid=grid,
        in_specs=[
            pl.BlockSpec((BLK, 128), lambda b: (b, 0)),
            pl.BlockSpec((BLK, 128), lambda b: (b, 0)),
            pl.BlockSpec((BLK, 128), lambda b: (b, 0)),
            pl.BlockSpec((BLK, 128), lambda b: (b, 0)),
            full((EMB, 64)),
            full((EMB, 64)),
            full((1, 64)),
            full((64, EMB)),
            full((1, EMB)),
            full((1, EMB)),
            full((1, EMB)),
            full((1, 1)),
        ],
        out_specs=pl.BlockSpec((BLK,), lambda b: (b,)),
        out_shape=jax.ShapeDtypeStruct((BATCH,), jnp.float32),
        compiler_params=pltpu.CompilerParams(
            dimension_semantics=("parallel",),
        ),
    )(Xu, Xi, mu_odd, mi_odd, W1a, W1b, b1, W2, b2, Wfg, Wfh, bf)


@jax.jit
def kernel(u, i, user_emb_gmf, item_emb_gmf, user_emb_mlp, item_emb_mlp,
           W1, b1, W2, b2, Wf, bf):
    u32 = jnp.asarray(u, jnp.int32)
    i32 = jnp.asarray(i, jnp.int32)

    M = _tc_mix(user_emb_gmf.T, item_emb_gmf.T,
                user_emb_mlp.T, item_emb_mlp.T)
    gu_idx = ((u32 >> 12) << 11) | (u32 & 2047)
    gi_idx = ((i32 >> 12) << 11) | (i32 & 2047)
    Xu, Xi = _sc_gather(gu_idx, gi_idx, M)

    ones = jnp.ones((1, 128), dtype=jnp.int32)
    mu_odd = ((u32 >> 11) & 1)[:, None] * ones
    mi_odd = ((i32 >> 11) & 1)[:, None] * ones

    W1a = W1[:EMB, :]
    W1b = W1[EMB:, :]
    Wfg = Wf[:EMB, 0].reshape(1, EMB)
    Wfh = Wf[EMB:, 0].reshape(1, EMB)
    out = _tc_dense(Xu, Xi, mu_odd, mi_odd, W1a, W1b, b1.reshape(1, 64), W2,
                    b2.reshape(1, EMB), Wfg, Wfh, bf.reshape(1, 1))
    return out


# speedup vs baseline: 5.0309x; 1.2169x over previous
import functools

import jax
import jax.numpy as jnp
from jax import lax
from jax.experimental import pallas as pl
from jax.experimental.pallas import tpu as pltpu
from jax.experimental.pallas import tpu_sc as plsc

BATCH = 16384
EMB = 32
NC = 2
NS = 16
NW = NC * NS
B_PER_W = BATCH // NW
CHUNK = 256

N_ROWS = 1000000
TBLK = 8192
NBLKS = 123
M_ROWS = NBLKS * TBLK // 2


def _tc_mix(ugT, igT, umT, imT):
    def body(a_r, b_r, c_r, d_r, o_r):
        x4 = jnp.concatenate([a_r[...], b_r[...], c_r[...], d_r[...]], axis=0)
        b = jax.lax.bitcast_convert_type(x4, jnp.int32)
        bb = (b + 32767 + ((b >> 16) & 1)) >> 16
        bbT = bb.T
        lo = bbT[0:TBLK // 2]
        hi = bbT[TBLK // 2:TBLK]
        o_r[...] = (lo & 65535) | (hi << 16)

    in_spec = pl.BlockSpec((EMB, TBLK), lambda b: (0, b))
    return pl.pallas_call(
        body,
        grid=(NBLKS,),
        in_specs=[in_spec] * 4,
        out_specs=pl.BlockSpec((TBLK // 2, 128), lambda b: (b, 0)),
        out_shape=jax.ShapeDtypeStruct((M_ROWS, 128), jnp.int32),
        compiler_params=pltpu.CompilerParams(
            dimension_semantics=("arbitrary",),
        ),
    )(ugT, igT, umT, imT)


def _sc_gather(u, i, M):
    mesh = plsc.VectorSubcoreMesh(core_axis_name="c", subcore_axis_name="s")
    out_t = tuple(jax.ShapeDtypeStruct((BATCH, 128), jnp.int32) for _ in range(2))

    @functools.partial(
        pl.kernel,
        mesh=mesh,
        out_type=out_t,
        scratch_types=[
            pltpu.VMEM((B_PER_W,), jnp.int32),
            pltpu.VMEM((B_PER_W,), jnp.int32),
            pltpu.VMEM((CHUNK, 128), jnp.int32),
            pltpu.VMEM((CHUNK, 128), jnp.int32),
            pltpu.SemaphoreType.DMA,
        ],
    )
    def k(u_hbm, i_hbm, m_hbm, o_u, o_i, uidx, iidx, bu, bi, sem):
        wid = lax.axis_index("s") * NC + lax.axis_index("c")
        base = wid * B_PER_W
        pltpu.sync_copy(u_hbm.at[pl.ds(base, B_PER_W)], uidx)
        pltpu.sync_copy(i_hbm.at[pl.ds(base, B_PER_W)], iidx)

        def chunk_body(c, carry):
            coff = c * CHUNK
            c0 = pltpu.async_copy(m_hbm.at[uidx.at[pl.ds(coff, CHUNK)]], bu, sem)
            c1 = pltpu.async_copy(m_hbm.at[iidx.at[pl.ds(coff, CHUNK)]], bi, sem)
            c0.wait()
            c1.wait()
            sl = pl.ds(base + coff, CHUNK)
            pltpu.sync_copy(bu, o_u.at[sl])
            pltpu.sync_copy(bi, o_i.at[sl])
            return carry

        lax.fori_loop(0, B_PER_W // CHUNK, chunk_body, 0)

    return k(u, i, M)


def _tc_dense(Xu, Xi, mu_odd, mi_odd, W1a, W1b, b1, W2, b2, Wfg, Wfh, bf):
    BLK = 2048
    grid = (BATCH // BLK,)

    def unpack(x_i32, odd):
        bits = jnp.where(odd != 0,
                         x_i32 & jnp.int32(-65536),
                         x_i32 << 16)
        return jax.lax.bitcast_convert_type(bits, jnp.float32)

    def body(xu_r, xi_r, mou_r, moi_r, W1a_r, W1b_r, b1_r, W2_r, b2_r,
             Wfg_r, Wfh_r, bf_r, out_r):
        xu = unpack(xu_r[...], mou_r[...])
        xi = unpack(xi_r[...], moi_r[...])
        gu = xu[:, 0:32]
        mu = xu[:, 64:96]
        gi = xi[:, 32:64]
        mi = xi[:, 96:128]
        h = jnp.dot(mu, W1a_r[...], preferred_element_type=jnp.float32)
        h = h + jnp.dot(mi, W1b_r[...], preferred_element_type=jnp.float32)
        h = jnp.maximum(h + b1_r[...], 0.0)
        h2 = jnp.dot(h, W2_r[...], preferred_element_type=jnp.float32)
        h2 = jnp.maximum(h2 + b2_r[...], 0.0)
        gmf = gu * gi
        acc = jnp.sum(gmf * Wfg_r[...], axis=1) + jnp.sum(h2 * Wfh_r[...], axis=1)
        out_r[...] = acc + bf_r[0, 0]

    full = lambda s: pl.BlockSpec(s, lambda b: (0, 0))
    return pl.pallas_call(
        body,
        grid=grid,
        in_specs=[
            pl.BlockSpec((BLK, 128), lambda b: (b, 0)),
            pl.BlockSpec((BLK, 128), lambda b: (b, 0)),
            pl.BlockSpec((BLK, 128), lambda b: (b, 0)),
            pl.BlockSpec((BLK, 128), lambda b: (b, 0)),
            full((EMB, 64)),
            full((EMB, 64)),
            full((1, 64)),
            full((64, EMB)),
            full((1, EMB)),
            full((1, EMB)),
            full((1, EMB)),
            full((1, 1)),
        ],
        out_specs=pl.BlockSpec((BLK,), lambda b: (b,)),
        out_shape=jax.ShapeDtypeStruct((BATCH,), jnp.float32),
        compiler_params=pltpu.CompilerParams(
            dimension_semantics=("parallel",),
        ),
    )(Xu, Xi, mu_odd, mi_odd, W1a, W1b, b1, W2, b2, Wfg, Wfh, bf)


@jax.jit
def kernel(u, i, user_emb_gmf, item_emb_gmf, user_emb_mlp, item_emb_mlp,
           W1, b1, W2, b2, Wf, bf):
    u32 = jnp.asarray(u, jnp.int32)
    i32 = jnp.asarray(i, jnp.int32)

    M = _tc_mix(user_emb_gmf.T, item_emb_gmf.T,
                user_emb_mlp.T, item_emb_mlp.T)
    gu_idx = ((u32 >> 13) << 12) | (u32 & 4095)
    gi_idx = ((i32 >> 13) << 12) | (i32 & 4095)
    Xu, Xi = _sc_gather(gu_idx, gi_idx, M)

    ones = jnp.ones((1, 128), dtype=jnp.int32)
    mu_odd = ((u32 >> 12) & 1)[:, None] * ones
    mi_odd = ((i32 >> 12) & 1)[:, None] * ones

    W1a = W1[:EMB, :]
    W1b = W1[EMB:, :]
    Wfg = Wf[:EMB, 0].reshape(1, EMB)
    Wfh = Wf[EMB:, 0].reshape(1, EMB)
    out = _tc_dense(Xu, Xi, mu_odd, mi_odd, W1a, W1b, b1.reshape(1, 64), W2,
                    b2.reshape(1, EMB), Wfg, Wfh, bf.reshape(1, 1))
    return out


# speedup vs baseline: 5.2825x; 1.0500x over previous
import functools

import jax
import jax.numpy as jnp
from jax import lax
from jax.experimental import pallas as pl
from jax.experimental.pallas import tpu as pltpu
from jax.experimental.pallas import tpu_sc as plsc

BATCH = 16384
EMB = 32
NC = 2
NS = 16
NW = NC * NS
B_PER_W = BATCH // NW
CHUNK = 256

N_ROWS = 1000000
TBLK = 16384
NBLKS = 62
M_ROWS = NBLKS * TBLK // 2


def _tc_mix(ugT, igT, umT, imT):
    def body(a_r, b_r, c_r, d_r, o_r):
        x4 = jnp.concatenate([a_r[...], b_r[...], c_r[...], d_r[...]], axis=0)
        b = jax.lax.bitcast_convert_type(x4, jnp.int32)
        bb = (b + 32767 + ((b >> 16) & 1)) >> 16
        bbT = bb.T
        lo = bbT[0:TBLK // 2]
        hi = bbT[TBLK // 2:TBLK]
        o_r[...] = (lo & 65535) | (hi << 16)

    in_spec = pl.BlockSpec((EMB, TBLK), lambda b: (0, b))
    return pl.pallas_call(
        body,
        grid=(NBLKS,),
        in_specs=[in_spec] * 4,
        out_specs=pl.BlockSpec((TBLK // 2, 128), lambda b: (b, 0)),
        out_shape=jax.ShapeDtypeStruct((M_ROWS, 128), jnp.int32),
        compiler_params=pltpu.CompilerParams(
            dimension_semantics=("arbitrary",),
        ),
    )(ugT, igT, umT, imT)


def _sc_gather(u, i, M):
    mesh = plsc.VectorSubcoreMesh(core_axis_name="c", subcore_axis_name="s")
    out_t = tuple(jax.ShapeDtypeStruct((BATCH, 128), jnp.int32) for _ in range(2))

    @functools.partial(
        pl.kernel,
        mesh=mesh,
        out_type=out_t,
        scratch_types=[
            pltpu.VMEM((B_PER_W,), jnp.int32),
            pltpu.VMEM((B_PER_W,), jnp.int32),
            pltpu.VMEM((CHUNK, 128), jnp.int32),
            pltpu.VMEM((CHUNK, 128), jnp.int32),
            pltpu.SemaphoreType.DMA,
        ],
    )
    def k(u_hbm, i_hbm, m_hbm, o_u, o_i, uidx, iidx, bu, bi, sem):
        wid = lax.axis_index("s") * NC + lax.axis_index("c")
        base = wid * B_PER_W
        pltpu.sync_copy(u_hbm.at[pl.ds(base, B_PER_W)], uidx)
        pltpu.sync_copy(i_hbm.at[pl.ds(base, B_PER_W)], iidx)

        def chunk_body(c, carry):
            coff = c * CHUNK
            c0 = pltpu.async_copy(m_hbm.at[uidx.at[pl.ds(coff, CHUNK)]], bu, sem)
            c1 = pltpu.async_copy(m_hbm.at[iidx.at[pl.ds(coff, CHUNK)]], bi, sem)
            c0.wait()
            c1.wait()
            sl = pl.ds(base + coff, CHUNK)
            pltpu.sync_copy(bu, o_u.at[sl])
            pltpu.sync_copy(bi, o_i.at[sl])
            return carry

        lax.fori_loop(0, B_PER_W // CHUNK, chunk_body, 0)

    return k(u, i, M)


def _tc_dense(Xu, Xi, mu_odd, mi_odd, W1a, W1b, b1, W2, b2, Wfg, Wfh, bf):
    BLK = 2048
    grid = (BATCH // BLK,)

    def unpack(x_i32, odd):
        bits = jnp.where(odd != 0,
                         x_i32 & jnp.int32(-65536),
                         x_i32 << 16)
        return jax.lax.bitcast_convert_type(bits, jnp.float32)

    def body(xu_r, xi_r, mou_r, moi_r, W1a_r, W1b_r, b1_r, W2_r, b2_r,
             Wfg_r, Wfh_r, bf_r, out_r):
        xu = unpack(xu_r[...], mou_r[...])
        xi = unpack(xi_r[...], moi_r[...])
        gu = xu[:, 0:32]
        mu = xu[:, 64:96]
        gi = xi[:, 32:64]
        mi = xi[:, 96:128]
        h = jnp.dot(mu, W1a_r[...], preferred_element_type=jnp.float32)
        h = h + jnp.dot(mi, W1b_r[...], preferred_element_type=jnp.float32)
        h = jnp.maximum(h + b1_r[...], 0.0)
        h2 = jnp.dot(h, W2_r[...], preferred_element_type=jnp.float32)
        h2 = jnp.maximum(h2 + b2_r[...], 0.0)
        gmf = gu * gi
        acc = jnp.sum(gmf * Wfg_r[...], axis=1) + jnp.sum(h2 * Wfh_r[...], axis=1)
        out_r[...] = acc + bf_r[0, 0]

    full = lambda s: pl.BlockSpec(s, lambda b: (0, 0))
    return pl.pallas_call(
        body,
        grid=grid,
        in_specs=[
            pl.BlockSpec((BLK, 128), lambda b: (b, 0)),
            pl.BlockSpec((BLK, 128), lambda b: (b, 0)),
            pl.BlockSpec((BLK, 128), lambda b: (b, 0)),
            pl.BlockSpec((BLK, 128), lambda b: (b, 0)),
            full((EMB, 64)),
            full((EMB, 64)),
            full((1, 64)),
            full((64, EMB)),
            full((1, EMB)),
            full((1, EMB)),
            full((1, EMB)),
            full((1, 1)),
        ],
        out_specs=pl.BlockSpec((BLK,), lambda b: (b,)),
        out_shape=jax.ShapeDtypeStruct((BATCH,), jnp.float32),
        compiler_params=pltpu.CompilerParams(
            dimension_semantics=("parallel",),
        ),
    )(Xu, Xi, mu_odd, mi_odd, W1a, W1b, b1, W2, b2, Wfg, Wfh, bf)


@jax.jit
def kernel(u, i, user_emb_gmf, item_emb_gmf, user_emb_mlp, item_emb_mlp,
           W1, b1, W2, b2, Wf, bf):
    u32 = jnp.asarray(u, jnp.int32)
    i32 = jnp.asarray(i, jnp.int32)

    M = _tc_mix(user_emb_gmf.T, item_emb_gmf.T,
                user_emb_mlp.T, item_emb_mlp.T)
    gu_idx = ((u32 >> 14) << 13) | (u32 & 8191)
    gi_idx = ((i32 >> 14) << 13) | (i32 & 8191)
    Xu, Xi = _sc_gather(gu_idx, gi_idx, M)

    ones = jnp.ones((1, 128), dtype=jnp.int32)
    mu_odd = ((u32 >> 13) & 1)[:, None] * ones
    mi_odd = ((i32 >> 13) & 1)[:, None] * ones

    W1a = W1[:EMB, :]
    W1b = W1[EMB:, :]
    Wfg = Wf[:EMB, 0].reshape(1, EMB)
    Wfh = Wf[EMB:, 0].reshape(1, EMB)
    out = _tc_dense(Xu, Xi, mu_odd, mi_odd, W1a, W1b, b1.reshape(1, 64), W2,
                    b2.reshape(1, EMB), Wfg, Wfh, bf.reshape(1, 1))
    return out


# speedup vs baseline: 5.3875x; 1.0199x over previous
import functools

import jax
import jax.numpy as jnp
from jax import lax
from jax.experimental import pallas as pl
from jax.experimental.pallas import tpu as pltpu
from jax.experimental.pallas import tpu_sc as plsc

BATCH = 16384
EMB = 32
NC = 2
NS = 16
NW = NC * NS
B_PER_W = BATCH // NW
CHUNK = 256

N_ROWS = 1000000
TBLK = 32768
NBLKS = 31
M_ROWS = NBLKS * TBLK // 2


def _tc_mix(ugT, igT, umT, imT):
    def body(a_r, b_r, c_r, d_r, o_r):
        x4 = jnp.concatenate([a_r[...], b_r[...], c_r[...], d_r[...]], axis=0)
        b = jax.lax.bitcast_convert_type(x4, jnp.int32)
        bb = (b + 32767 + ((b >> 16) & 1)) >> 16
        bbT = bb.T
        lo = bbT[0:TBLK // 2]
        hi = bbT[TBLK // 2:TBLK]
        o_r[...] = (lo & 65535) | (hi << 16)

    in_spec = pl.BlockSpec((EMB, TBLK), lambda b: (0, b))
    return pl.pallas_call(
        body,
        grid=(NBLKS,),
        in_specs=[in_spec] * 4,
        out_specs=pl.BlockSpec((TBLK // 2, 128), lambda b: (b, 0)),
        out_shape=jax.ShapeDtypeStruct((M_ROWS, 128), jnp.int32),
        compiler_params=pltpu.CompilerParams(
            dimension_semantics=("arbitrary",),
        ),
    )(ugT, igT, umT, imT)


def _sc_gather(u, i, M):
    mesh = plsc.VectorSubcoreMesh(core_axis_name="c", subcore_axis_name="s")
    out_t = tuple(jax.ShapeDtypeStruct((BATCH, 128), jnp.int32) for _ in range(2))

    @functools.partial(
        pl.kernel,
        mesh=mesh,
        out_type=out_t,
        scratch_types=[
            pltpu.VMEM((B_PER_W,), jnp.int32),
            pltpu.VMEM((B_PER_W,), jnp.int32),
            pltpu.VMEM((CHUNK, 128), jnp.int32),
            pltpu.VMEM((CHUNK, 128), jnp.int32),
            pltpu.SemaphoreType.DMA,
        ],
    )
    def k(u_hbm, i_hbm, m_hbm, o_u, o_i, uidx, iidx, bu, bi, sem):
        wid = lax.axis_index("s") * NC + lax.axis_index("c")
        base = wid * B_PER_W
        pltpu.sync_copy(u_hbm.at[pl.ds(base, B_PER_W)], uidx)
        pltpu.sync_copy(i_hbm.at[pl.ds(base, B_PER_W)], iidx)

        def chunk_body(c, carry):
            coff = c * CHUNK
            c0 = pltpu.async_copy(m_hbm.at[uidx.at[pl.ds(coff, CHUNK)]], bu, sem)
            c1 = pltpu.async_copy(m_hbm.at[iidx.at[pl.ds(coff, CHUNK)]], bi, sem)
            c0.wait()
            c1.wait()
            sl = pl.ds(base + coff, CHUNK)
            pltpu.sync_copy(bu, o_u.at[sl])
            pltpu.sync_copy(bi, o_i.at[sl])
            return carry

        lax.fori_loop(0, B_PER_W // CHUNK, chunk_body, 0)

    return k(u, i, M)


def _tc_dense(Xu, Xi, mu_odd, mi_odd, W1a, W1b, b1, W2, b2, Wfg, Wfh, bf):
    BLK = 2048
    grid = (BATCH // BLK,)

    def unpack(x_i32, odd):
        bits = jnp.where(odd != 0,
                         x_i32 & jnp.int32(-65536),
                         x_i32 << 16)
        return jax.lax.bitcast_convert_type(bits, jnp.float32)

    def body(xu_r, xi_r, mou_r, moi_r, W1a_r, W1b_r, b1_r, W2_r, b2_r,
             Wfg_r, Wfh_r, bf_r, out_r):
        xu = unpack(xu_r[...], mou_r[...])
        xi = unpack(xi_r[...], moi_r[...])
        gu = xu[:, 0:32]
        mu = xu[:, 64:96]
        gi = xi[:, 32:64]
        mi = xi[:, 96:128]
        h = jnp.dot(mu, W1a_r[...], preferred_element_type=jnp.float32)
        h = h + jnp.dot(mi, W1b_r[...], preferred_element_type=jnp.float32)
        h = jnp.maximum(h + b1_r[...], 0.0)
        h2 = jnp.dot(h, W2_r[...], preferred_element_type=jnp.float32)
        h2 = jnp.maximum(h2 + b2_r[...], 0.0)
        gmf = gu * gi
        acc = jnp.sum(gmf * Wfg_r[...], axis=1) + jnp.sum(h2 * Wfh_r[...], axis=1)
        out_r[...] = acc + bf_r[0, 0]

    full = lambda s: pl.BlockSpec(s, lambda b: (0, 0))
    return pl.pallas_call(
        body,
        grid=grid,
        in_specs=[
            pl.BlockSpec((BLK, 128), lambda b: (b, 0)),
            pl.BlockSpec((BLK, 128), lambda b: (b, 0)),
            pl.BlockSpec((BLK, 128), lambda b: (b, 0)),
            pl.BlockSpec((BLK, 128), lambda b: (b, 0)),
            full((EMB, 64)),
            full((EMB, 64)),
            full((1, 64)),
            full((64, EMB)),
            full((1, EMB)),
            full((1, EMB)),
            full((1, EMB)),
            full((1, 1)),
        ],
        out_specs=pl.BlockSpec((BLK,), lambda b: (b,)),
        out_shape=jax.ShapeDtypeStruct((BATCH,), jnp.float32),
        compiler_params=pltpu.CompilerParams(
            dimension_semantics=("parallel",),
        ),
    )(Xu, Xi, mu_odd, mi_odd, W1a, W1b, b1, W2, b2, Wfg, Wfh, bf)


@jax.jit
def kernel(u, i, user_emb_gmf, item_emb_gmf, user_emb_mlp, item_emb_mlp,
           W1, b1, W2, b2, Wf, bf):
    u32 = jnp.asarray(u, jnp.int32)
    i32 = jnp.asarray(i, jnp.int32)

    M = _tc_mix(user_emb_gmf.T, item_emb_gmf.T,
                user_emb_mlp.T, item_emb_mlp.T)
    gu_idx = ((u32 >> 15) << 14) | (u32 & 16383)
    gi_idx = ((i32 >> 15) << 14) | (i32 & 16383)
    Xu, Xi = _sc_gather(gu_idx, gi_idx, M)

    ones = jnp.ones((1, 128), dtype=jnp.int32)
    mu_odd = ((u32 >> 14) & 1)[:, None] * ones
    mi_odd = ((i32 >> 14) & 1)[:, None] * ones

    W1a = W1[:EMB, :]
    W1b = W1[EMB:, :]
    Wfg = Wf[:EMB, 0].reshape(1, EMB)
    Wfh = Wf[EMB:, 0].reshape(1, EMB)
    out = _tc_dense(Xu, Xi, mu_odd, mi_odd, W1a, W1b, b1.reshape(1, 64), W2,
                    b2.reshape(1, EMB), Wfg, Wfh, bf.reshape(1, 1))
    return out
